# Initial kernel scaffold; baseline (speedup 1.0000x reference)
import jax
import jax.numpy as jnp
from jax.experimental import pallas as pl


def kernel(x, edge_index, W1, b1, W2, b2):
    raise NotImplementedError("write your pallas kernel here")


# speedup vs baseline: 24.5589x; 24.5589x over previous
import functools

import jax
import jax.numpy as jnp
from jax import lax
from jax.experimental import pallas as pl
from jax.experimental.pallas import tpu as pltpu
from jax.experimental.pallas import tpu_sc as plsc

N = 10000
E = 320000
F_IN = 128
HID = 16
C = 40

NC = 2
NS = 16
NW = NC * NS
EPW = E // NW
CH = 80
NCHUNK = EPW // CH
N_PAD = 10240
RPS = N_PAD // NS
DEGW = 16


def _sc_mesh():
  return plsc.VectorSubcoreMesh(core_axis_name="c", subcore_axis_name="s")


def _deg_kernel():
  @functools.partial(
      pl.kernel,
      out_type=jax.ShapeDtypeStruct((NC, N_PAD, DEGW), jnp.float32),
      mesh=_sc_mesh(),
      scratch_types=[
          pltpu.VMEM((NCHUNK, CH), jnp.int32),
          pltpu.VMEM((CH, DEGW), jnp.float32),
          pltpu.VMEM_SHARED((N_PAD, DEGW), jnp.float32),
      ],
      compiler_params=pltpu.CompilerParams(use_tc_tiling_on_sc=False),
  )
  def k(dst_hbm, ones_hbm, zeros_hbm, out_hbm, dst_v, ones_v, acc):
    cid = lax.axis_index("c")
    sid = lax.axis_index("s")
    wid = cid * NS + sid
    pltpu.sync_copy(zeros_hbm, acc.at[pl.ds(sid * RPS, RPS)])
    pltpu.sync_copy(dst_hbm.at[wid], dst_v)
    pltpu.sync_copy(ones_hbm, ones_v)
    plsc.subcore_barrier()

    def body(j, carry):
      pltpu.sync_copy(ones_v, acc.at[dst_v.at[j]], add=True)
      return carry

    lax.fori_loop(0, NCHUNK, body, 0)
    plsc.subcore_barrier()
    pltpu.sync_copy(acc.at[pl.ds(sid * RPS, RPS)],
                    out_hbm.at[cid, pl.ds(sid * RPS, RPS)])

  return k


def _agg_kernel(feat):
  @functools.partial(
      pl.kernel,
      out_type=jax.ShapeDtypeStruct((NC, N_PAD, feat), jnp.float32),
      mesh=_sc_mesh(),
      scratch_types=[
          pltpu.VMEM((NCHUNK, CH), jnp.int32),
          pltpu.VMEM((NCHUNK, CH), jnp.int32),
          pltpu.VMEM((CH, feat), jnp.float32),
          pltpu.VMEM_SHARED((N_PAD, feat), jnp.float32),
          pltpu.SemaphoreType.DMA,
      ],
      compiler_params=pltpu.CompilerParams(use_tc_tiling_on_sc=False),
  )
  def k(xs_hbm, src_hbm, dst_hbm, zeros_hbm, out_hbm,
        src_v, dst_v, rows_v, acc, gsem):
    cid = lax.axis_index("c")
    sid = lax.axis_index("s")
    wid = cid * NS + sid
    pltpu.sync_copy(zeros_hbm, acc.at[pl.ds(sid * RPS, RPS)])
    pltpu.sync_copy(src_hbm.at[wid], src_v)
    pltpu.sync_copy(dst_hbm.at[wid], dst_v)
    plsc.subcore_barrier()

    def body(j, carry):
      pltpu.async_copy(xs_hbm.at[src_v.at[j]], rows_v, gsem).wait()
      pltpu.sync_copy(rows_v, acc.at[dst_v.at[j]], add=True)
      return carry

    lax.fori_loop(0, NCHUNK, body, 0)
    plsc.subcore_barrier()
    pltpu.sync_copy(acc.at[pl.ds(sid * RPS, RPS)],
                    out_hbm.at[cid, pl.ds(sid * RPS, RPS)])

  return k


_ROWS = 1000


def _dinv_of(deg_ref0, deg_ref1):
  return lax.rsqrt(deg_ref0[:, 0:1] + deg_ref1[:, 0:1] + 1.0)


def _tc1_body(x_ref, w1_ref, d0_ref, d1_ref, o_ref):
  dinv = _dinv_of(d0_ref, d1_ref)
  xw = jnp.dot(x_ref[...], w1_ref[...], preferred_element_type=jnp.float32)
  o_ref[...] = xw * dinv


def _tc2_body(a0_ref, a1_ref, xs1_ref, d0_ref, d1_ref, w2_ref, b1_ref, o_ref):
  dinv = _dinv_of(d0_ref, d1_ref)
  h = a0_ref[...] + a1_ref[...] + xs1_ref[...]
  h = jnp.maximum(h * dinv + b1_ref[...], 0.0)
  o_ref[...] = jnp.dot(h, w2_ref[...],
                       preferred_element_type=jnp.float32) * dinv


def _tc3_body(a0_ref, a1_ref, xs2_ref, d0_ref, d1_ref, b2_ref, o_ref):
  dinv = _dinv_of(d0_ref, d1_ref)
  o = (a0_ref[...] + a1_ref[...] + xs2_ref[...]) * dinv + b2_ref[...]
  m = jnp.max(o, axis=1, keepdims=True)
  lse = jnp.log(jnp.sum(jnp.exp(o - m), axis=1, keepdims=True)) + m
  o_ref[...] = o - lse


def _row_spec(w):
  return pl.BlockSpec((_ROWS, w), lambda i: (i, 0))


def _full_spec(shape):
  return pl.BlockSpec(shape, lambda i: tuple(0 for _ in shape))


@jax.jit
def kernel(x, edge_index, W1, b1, W2, b2):
  src3 = edge_index[0].reshape(NW, NCHUNK, CH)
  dst3 = edge_index[1].reshape(NW, NCHUNK, CH)
  ones = jnp.ones((CH, DEGW), jnp.float32)
  zeros_d = jnp.zeros((RPS, DEGW), jnp.float32)
  zeros_1 = jnp.zeros((RPS, HID), jnp.float32)
  zeros_2 = jnp.zeros((RPS, C), jnp.float32)

  degp = _deg_kernel()(dst3, ones, zeros_d)
  d0 = degp[0, :N]
  d1 = degp[1, :N]

  grid = (N // _ROWS,)
  xs1 = pl.pallas_call(
      _tc1_body,
      grid=grid,
      in_specs=[_row_spec(F_IN), _full_spec((F_IN, HID)),
                _row_spec(DEGW), _row_spec(DEGW)],
      out_specs=_row_spec(HID),
      out_shape=jax.ShapeDtypeStruct((N, HID), jnp.float32),
  )(x, W1, d0, d1)

  agg1 = _agg_kernel(HID)(xs1, src3, dst3, zeros_1)

  xs2 = pl.pallas_call(
      _tc2_body,
      grid=grid,
      in_specs=[_row_spec(HID), _row_spec(HID), _row_spec(HID),
                _row_spec(DEGW), _row_spec(DEGW),
                _full_spec((HID, C)), _full_spec((1, HID))],
      out_specs=_row_spec(C),
      out_shape=jax.ShapeDtypeStruct((N, C), jnp.float32),
  )(agg1[0, :N], agg1[1, :N], xs1, d0, d1, W2, b1.reshape(1, HID))

  agg2 = _agg_kernel(C)(xs2, src3, dst3, zeros_2)

  out = pl.pallas_call(
      _tc3_body,
      grid=grid,
      in_specs=[_row_spec(C), _row_spec(C), _row_spec(C),
                _row_spec(DEGW), _row_spec(DEGW), _full_spec((1, C))],
      out_specs=_row_spec(C),
      out_shape=jax.ShapeDtypeStruct((N, C), jnp.float32),
  )(agg2[0, :N], agg2[1, :N], xs2, d0, d1, b2.reshape(1, C))

  return out


# speedup vs baseline: 26.2325x; 1.0681x over previous
import functools

import jax
import jax.numpy as jnp
from jax import lax
from jax.experimental import pallas as pl
from jax.experimental.pallas import tpu as pltpu
from jax.experimental.pallas import tpu_sc as plsc

N = 10000
E = 320000
F_IN = 128
HID = 16
C = 40

NC = 2
NS = 16
NW = NC * NS
CH = 128
NCHUNK = 80
EPW = NCHUNK * CH
E_PAD = NW * EPW
N_PAD = 10240
RPS = N_PAD // NS
DUMMY = N_PAD - 8
DEGW = 16
NBUF = 4

_SC_PARAMS = pltpu.CompilerParams(use_tc_tiling_on_sc=False)


def _sc_mesh():
  return plsc.VectorSubcoreMesh(core_axis_name="c", subcore_axis_name="s")


def _deg_kernel():
  @functools.partial(
      pl.kernel,
      out_type=jax.ShapeDtypeStruct((NC, N_PAD, DEGW), jnp.float32),
      mesh=_sc_mesh(),
      scratch_types=[
          pltpu.VMEM((NCHUNK, CH), jnp.int32),
          pltpu.VMEM((CH, DEGW), jnp.float32),
          pltpu.VMEM_SHARED((N_PAD, DEGW), jnp.float32),
      ],
      compiler_params=_SC_PARAMS,
  )
  def k(dst_hbm, ones_hbm, zeros_hbm, out_hbm, dst_v, ones_v, acc):
    cid = lax.axis_index("c")
    sid = lax.axis_index("s")
    wid = cid * NS + sid
    pltpu.sync_copy(zeros_hbm, acc.at[pl.ds(sid * RPS, RPS)])
    pltpu.sync_copy(dst_hbm.at[wid], dst_v)
    pltpu.sync_copy(ones_hbm, ones_v)
    plsc.subcore_barrier()

    def body(j, carry):
      pltpu.sync_copy(ones_v, acc.at[dst_v.at[j]], add=True)
      return carry

    lax.fori_loop(0, NCHUNK, body, 0)
    plsc.subcore_barrier()
    pltpu.sync_copy(acc.at[pl.ds(sid * RPS, RPS)],
                    out_hbm.at[cid, pl.ds(sid * RPS, RPS)])

  return k


def _agg_kernel(feat):
  @functools.partial(
      pl.kernel,
      out_type=jax.ShapeDtypeStruct((NC, N_PAD, feat), jnp.float32),
      mesh=_sc_mesh(),
      scratch_types=[
          pltpu.VMEM((NCHUNK, CH), jnp.int32),
          pltpu.VMEM((NCHUNK, CH), jnp.int32),
          pltpu.VMEM((NBUF, CH, feat), jnp.float32),
          pltpu.VMEM_SHARED((N_PAD, feat), jnp.float32),
          pltpu.SemaphoreType.DMA((NBUF,)),
      ],
      compiler_params=_SC_PARAMS,
  )
  def k(xs_hbm, src_hbm, dst_hbm, zeros_hbm, out_hbm,
        src_v, dst_v, bufs, acc, gsems):
    cid = lax.axis_index("c")
    sid = lax.axis_index("s")
    wid = cid * NS + sid
    pltpu.sync_copy(zeros_hbm, acc.at[pl.ds(sid * RPS, RPS)])
    pltpu.sync_copy(src_hbm.at[wid], src_v)
    pltpu.sync_copy(dst_hbm.at[wid], dst_v)
    plsc.subcore_barrier()

    for b in range(NBUF):
      pltpu.async_copy(xs_hbm.at[src_v.at[b]], bufs.at[b], gsems.at[b])

    ngrp = NCHUNK // NBUF

    def grp(p, carry):
      for b in range(NBUF):
        j = p * NBUF + b
        pltpu.make_async_copy(xs_hbm.at[src_v.at[j]], bufs.at[b],
                              gsems.at[b]).wait()
        pltpu.sync_copy(bufs.at[b], acc.at[dst_v.at[j]], add=True)

        @pl.when(p < ngrp - 1)
        def _():
          pltpu.async_copy(xs_hbm.at[src_v.at[j + NBUF]], bufs.at[b],
                           gsems.at[b])

      return carry

    lax.fori_loop(0, ngrp, grp, 0)
    plsc.subcore_barrier()
    pltpu.sync_copy(acc.at[pl.ds(sid * RPS, RPS)],
                    out_hbm.at[cid, pl.ds(sid * RPS, RPS)])

  return k


_ROWS = 1000


def _dinv_of(deg_ref0, deg_ref1):
  return lax.rsqrt(deg_ref0[:, 0:1] + deg_ref1[:, 0:1] + 1.0)


def _tc1_body(x_ref, w1_ref, d0_ref, d1_ref, o_ref):
  dinv = _dinv_of(d0_ref, d1_ref)
  xw = jnp.dot(x_ref[...], w1_ref[...], preferred_element_type=jnp.float32)
  o_ref[...] = xw * dinv


def _tc2_body(a0_ref, a1_ref, xs1_ref, d0_ref, d1_ref, w2_ref, b1_ref, o_ref):
  dinv = _dinv_of(d0_ref, d1_ref)
  h = a0_ref[...] + a1_ref[...] + xs1_ref[...]
  h = jnp.maximum(h * dinv + b1_ref[...], 0.0)
  o_ref[...] = jnp.dot(h, w2_ref[...],
                       preferred_element_type=jnp.float32) * dinv


def _tc3_body(a0_ref, a1_ref, xs2_ref, d0_ref, d1_ref, b2_ref, o_ref):
  dinv = _dinv_of(d0_ref, d1_ref)
  o = (a0_ref[...] + a1_ref[...] + xs2_ref[...]) * dinv + b2_ref[...]
  m = jnp.max(o, axis=1, keepdims=True)
  lse = jnp.log(jnp.sum(jnp.exp(o - m), axis=1, keepdims=True)) + m
  o_ref[...] = o - lse


def _row_spec(w):
  return pl.BlockSpec((_ROWS, w), lambda i: (i, 0))


def _full_spec(shape):
  return pl.BlockSpec(shape, lambda i: tuple(0 for _ in shape))


@jax.jit
def kernel(x, edge_index, W1, b1, W2, b2):
  npad = E_PAD - E
  src3 = jnp.concatenate(
      [edge_index[0], jnp.zeros((npad,), jnp.int32)]).reshape(NW, NCHUNK, CH)
  dst3 = jnp.concatenate(
      [edge_index[1], jnp.full((npad,), DUMMY, jnp.int32)]).reshape(
          NW, NCHUNK, CH)
  ones = jnp.ones((CH, DEGW), jnp.float32)
  zeros_d = jnp.zeros((RPS, DEGW), jnp.float32)
  zeros_1 = jnp.zeros((RPS, HID), jnp.float32)
  zeros_2 = jnp.zeros((RPS, C), jnp.float32)

  degp = _deg_kernel()(dst3, ones, zeros_d)
  d0 = degp[0, :N]
  d1 = degp[1, :N]

  grid = (N // _ROWS,)
  xs1 = pl.pallas_call(
      _tc1_body,
      grid=grid,
      in_specs=[_row_spec(F_IN), _full_spec((F_IN, HID)),
                _row_spec(DEGW), _row_spec(DEGW)],
      out_specs=_row_spec(HID),
      out_shape=jax.ShapeDtypeStruct((N, HID), jnp.float32),
  )(x, W1, d0, d1)

  agg1 = _agg_kernel(HID)(xs1, src3, dst3, zeros_1)

  xs2 = pl.pallas_call(
      _tc2_body,
      grid=grid,
      in_specs=[_row_spec(HID), _row_spec(HID), _row_spec(HID),
                _row_spec(DEGW), _row_spec(DEGW),
                _full_spec((HID, C)), _full_spec((1, HID))],
      out_specs=_row_spec(C),
      out_shape=jax.ShapeDtypeStruct((N, C), jnp.float32),
  )(agg1[0, :N], agg1[1, :N], xs1, d0, d1, W2, b1.reshape(1, HID))

  agg2 = _agg_kernel(C)(xs2, src3, dst3, zeros_2)

  out = pl.pallas_call(
      _tc3_body,
      grid=grid,
      in_specs=[_row_spec(C), _row_spec(C), _row_spec(C),
                _row_spec(DEGW), _row_spec(DEGW), _full_spec((1, C))],
      out_specs=_row_spec(C),
      out_shape=jax.ShapeDtypeStruct((N, C), jnp.float32),
  )(agg2[0, :N], agg2[1, :N], xs2, d0, d1, b2.reshape(1, C))

  return out
